# baseline (speedup 1.0000x reference)
import functools

import jax
import jax.numpy as jnp
from jax import lax
from jax.experimental import pallas as pl
from jax.experimental.pallas import tpu as pltpu


def _round_up(x, m):
    return ((x + m - 1) // m) * m


def _prenorm_matmul_kernel(x_ref, g_ref, b_ref, w_hbm, bias_ref, o_ref,
                           w_vmem, sems, *, eps, true_dim, n_chunks):
    j = pl.program_id(1)
    kp = x_ref.shape[-1]
    np_ = o_ref.shape[-1]
    cn = np_ // n_chunks

    @pl.when(j == 0)
    def _start_w_copies():
        for t in range(n_chunks):
            sl = slice(t * cn, (t + 1) * cn)
            pltpu.make_async_copy(
                w_hbm.at[:, sl], w_vmem.at[:, sl], sems.at[t]).start()

    x = x_ref[...].astype(jnp.float32)
    inv_d = 1.0 / float(true_dim)
    mean = jnp.sum(x, axis=-1, keepdims=True) * inv_d
    xc = x - mean
    if kp != true_dim:
        mask = lax.broadcasted_iota(jnp.int32, (1, kp), 1) < true_dim
        xc = jnp.where(mask, xc, 0.0)
    var = jnp.sum(xc * xc, axis=-1, keepdims=True) * inv_d
    inv = lax.rsqrt(var + eps)
    y = xc * inv * g_ref[...].astype(jnp.float32) + b_ref[...].astype(jnp.float32)
    y = y.astype(jnp.bfloat16)

    bias = bias_ref[...].astype(jnp.float32)
    for t in range(n_chunks):
        sl = slice(t * cn, (t + 1) * cn)

        @pl.when(j == 0)
        def _wait_chunk(t=t, sl=sl):
            pltpu.make_async_copy(
                w_vmem.at[:, sl], w_vmem.at[:, sl], sems.at[t]).wait()

        acc = jnp.dot(y, w_vmem[:, sl].astype(jnp.bfloat16),
                      preferred_element_type=jnp.float32)
        o_ref[:, sl] = (acc + bias[:, sl]).astype(o_ref.dtype)


def kernel(x, gamma, beta, w, b):
    eps = 1e-5
    orig_shape = x.shape
    din = orig_shape[-1]
    dout = w.shape[1]
    x2 = x.reshape(-1, din)
    rows = x2.shape[0]

    kp = max(_round_up(din, 128), 128)
    np_ = max(_round_up(dout, 128), 128)

    tm = min(512, _round_up(rows, 8))
    rows_p = _round_up(rows, tm)
    m_tiles = rows_p // tm

    n_cores = 2 if m_tiles % 2 == 0 else 1
    inner = m_tiles // n_cores

    n_chunks = 4
    while np_ % (n_chunks * 128) != 0 and n_chunks > 1:
        n_chunks //= 2

    x_p = jnp.pad(x2, ((0, rows_p - rows), (0, kp - din)))
    g_p = jnp.pad(gamma.reshape(1, din), ((0, 0), (0, kp - din)))
    b_p = jnp.pad(beta.reshape(1, din), ((0, 0), (0, kp - din)))
    w_p = jnp.pad(w, ((0, kp - din), (0, np_ - dout)))
    bias_p = jnp.pad(b.reshape(1, dout), ((0, 0), (0, np_ - dout)))

    cost = pl.CostEstimate(
        flops=2 * rows_p * kp * np_ + 8 * rows_p * kp,
        transcendentals=rows_p,
        bytes_accessed=rows_p * kp * 4 + kp * np_ * 4 + rows_p * np_ * 4,
    )

    out = pl.pallas_call(
        functools.partial(_prenorm_matmul_kernel, eps=eps, true_dim=din,
                          n_chunks=n_chunks),
        out_shape=jax.ShapeDtypeStruct((rows_p, np_), x.dtype),
        grid_spec=pltpu.PrefetchScalarGridSpec(
            num_scalar_prefetch=0,
            grid=(n_cores, inner),
            in_specs=[
                pl.BlockSpec((tm, kp), lambda c, j, inner=inner: (c * inner + j, 0)),
                pl.BlockSpec((1, kp), lambda c, j: (0, 0)),
                pl.BlockSpec((1, kp), lambda c, j: (0, 0)),
                pl.BlockSpec(memory_space=pl.ANY),
                pl.BlockSpec((1, np_), lambda c, j: (0, 0)),
            ],
            out_specs=pl.BlockSpec(
                (tm, np_), lambda c, j, inner=inner: (c * inner + j, 0)),
            scratch_shapes=[
                pltpu.VMEM((kp, np_), w.dtype),
                pltpu.SemaphoreType.DMA((n_chunks,)),
            ],
        ),
        compiler_params=pltpu.CompilerParams(
            dimension_semantics=("parallel", "arbitrary"),
            vmem_limit_bytes=60 * 1024 * 1024,
        ),
        cost_estimate=cost,
    )(x_p, g_p, b_p, w_p, bias_p)
    return out[:rows, :dout].reshape(orig_shape[:-1] + (dout,))


# speedup vs baseline: 1.0724x; 1.0724x over previous
import functools

import jax
import jax.numpy as jnp
from jax import lax
from jax.experimental import pallas as pl
from jax.experimental.pallas import tpu as pltpu


def _round_up(x, m):
    return ((x + m - 1) // m) * m


def _prenorm_matmul_kernel(x_ref, g_ref, b_ref, w_hbm, bias_ref, o_ref,
                           w_vmem, sems, *, eps, true_dim, n_chunks):
    j = pl.program_id(1)
    kp = x_ref.shape[-1]
    np_ = o_ref.shape[-1]
    cn = np_ // n_chunks

    @pl.when(j == 0)
    def _start_w_copies():
        for t in range(n_chunks):
            sl = slice(t * cn, (t + 1) * cn)
            pltpu.make_async_copy(
                w_hbm.at[:, sl], w_vmem.at[:, sl], sems.at[t]).start()

    x = x_ref[...].astype(jnp.float32)
    inv_d = 1.0 / float(true_dim)
    mean = jnp.sum(x, axis=-1, keepdims=True) * inv_d
    xc = x - mean
    if kp != true_dim:
        mask = lax.broadcasted_iota(jnp.int32, (1, kp), 1) < true_dim
        xc = jnp.where(mask, xc, 0.0)
    var = jnp.sum(xc * xc, axis=-1, keepdims=True) * inv_d
    inv = lax.rsqrt(var + eps)
    y = xc * inv * g_ref[...].astype(jnp.float32) + b_ref[...].astype(jnp.float32)
    y = y.astype(jnp.bfloat16)

    bias = bias_ref[...].astype(jnp.float32)

    @pl.when(j == 0)
    def _first_step():
        for t in range(n_chunks):
            sl = slice(t * cn, (t + 1) * cn)
            pltpu.make_async_copy(
                w_vmem.at[:, sl], w_vmem.at[:, sl], sems.at[t]).wait()
            acc = jnp.dot(y, w_vmem[:, sl].astype(jnp.bfloat16),
                          preferred_element_type=jnp.float32)
            o_ref[:, sl] = (acc + bias[:, sl]).astype(o_ref.dtype)

    @pl.when(j != 0)
    def _steady_step():
        acc = jnp.dot(y, w_vmem[...].astype(jnp.bfloat16),
                      preferred_element_type=jnp.float32)
        o_ref[...] = (acc + bias).astype(o_ref.dtype)


def kernel(x, gamma, beta, w, b):
    eps = 1e-5
    orig_shape = x.shape
    din = orig_shape[-1]
    dout = w.shape[1]
    x2 = x.reshape(-1, din)
    rows = x2.shape[0]

    kp = max(_round_up(din, 128), 128)
    np_ = max(_round_up(dout, 128), 128)

    tm = min(512, _round_up(rows, 8))
    rows_p = _round_up(rows, tm)
    m_tiles = rows_p // tm

    n_cores = 2 if m_tiles % 2 == 0 else 1
    inner = m_tiles // n_cores

    n_chunks = 4
    while np_ % (n_chunks * 128) != 0 and n_chunks > 1:
        n_chunks //= 2

    x_p = jnp.pad(x2, ((0, rows_p - rows), (0, kp - din)))
    g_p = jnp.pad(gamma.reshape(1, din), ((0, 0), (0, kp - din)))
    b_p = jnp.pad(beta.reshape(1, din), ((0, 0), (0, kp - din)))
    w_p = jnp.pad(w, ((0, kp - din), (0, np_ - dout)))
    bias_p = jnp.pad(b.reshape(1, dout), ((0, 0), (0, np_ - dout)))

    cost = pl.CostEstimate(
        flops=2 * rows_p * kp * np_ + 8 * rows_p * kp,
        transcendentals=rows_p,
        bytes_accessed=rows_p * kp * 4 + kp * np_ * 4 + rows_p * np_ * 4,
    )

    out = pl.pallas_call(
        functools.partial(_prenorm_matmul_kernel, eps=eps, true_dim=din,
                          n_chunks=n_chunks),
        out_shape=jax.ShapeDtypeStruct((rows_p, np_), x.dtype),
        grid_spec=pltpu.PrefetchScalarGridSpec(
            num_scalar_prefetch=0,
            grid=(n_cores, inner),
            in_specs=[
                pl.BlockSpec((tm, kp), lambda c, j, inner=inner: (c * inner + j, 0)),
                pl.BlockSpec((1, kp), lambda c, j: (0, 0)),
                pl.BlockSpec((1, kp), lambda c, j: (0, 0)),
                pl.BlockSpec(memory_space=pl.ANY),
                pl.BlockSpec((1, np_), lambda c, j: (0, 0)),
            ],
            out_specs=pl.BlockSpec(
                (tm, np_), lambda c, j, inner=inner: (c * inner + j, 0)),
            scratch_shapes=[
                pltpu.VMEM((kp, np_), w.dtype),
                pltpu.SemaphoreType.DMA((n_chunks,)),
            ],
        ),
        compiler_params=pltpu.CompilerParams(
            dimension_semantics=("parallel", "arbitrary"),
            vmem_limit_bytes=60 * 1024 * 1024,
        ),
        cost_estimate=cost,
    )(x_p, g_p, b_p, w_p, bias_p)
    return out[:rows, :dout].reshape(orig_shape[:-1] + (dout,))


# speedup vs baseline: 1.1879x; 1.1077x over previous
import functools

import jax
import jax.numpy as jnp
from jax import lax
from jax.experimental import pallas as pl
from jax.experimental.pallas import tpu as pltpu


def _round_up(x, m):
    return ((x + m - 1) // m) * m


def _prenorm_matmul_kernel(x_ref, g_ref, b_ref, w_ref, bias_ref, o_ref,
                           *, eps, true_dim):
    x = x_ref[...].astype(jnp.float32)
    kp = x.shape[-1]
    inv_d = 1.0 / float(true_dim)
    mean = jnp.sum(x, axis=-1, keepdims=True) * inv_d
    xc = x - mean
    if kp != true_dim:
        mask = lax.broadcasted_iota(jnp.int32, (1, kp), 1) < true_dim
        xc = jnp.where(mask, xc, 0.0)
    var = jnp.sum(xc * xc, axis=-1, keepdims=True) * inv_d
    inv = lax.rsqrt(var + eps)
    y = xc * inv * g_ref[...].astype(jnp.float32) + b_ref[...].astype(jnp.float32)
    y = y.astype(jnp.bfloat16)
    acc = jnp.dot(y, w_ref[...].astype(jnp.bfloat16),
                  preferred_element_type=jnp.float32)
    o_ref[...] = (acc + bias_ref[...].astype(jnp.float32)).astype(o_ref.dtype)


def kernel(x, gamma, beta, w, b):
    eps = 1e-5
    orig_shape = x.shape
    din = orig_shape[-1]
    dout = w.shape[1]
    x2 = x.reshape(-1, din)
    rows = x2.shape[0]

    kp = max(_round_up(din, 128), 128)
    np_ = max(_round_up(dout, 128), 128)

    tm = min(512, _round_up(rows, 8))
    rows_p = _round_up(rows, tm)
    m_tiles = rows_p // tm
    n_cores = 2 if m_tiles % 2 == 0 else 1
    inner = m_tiles // n_cores

    x_p = jnp.pad(x2, ((0, rows_p - rows), (0, kp - din)))
    g_p = jnp.pad(gamma.reshape(1, din), ((0, 0), (0, kp - din)))
    b_p = jnp.pad(beta.reshape(1, din), ((0, 0), (0, kp - din)))
    w_p = jnp.pad(w, ((0, kp - din), (0, np_ - dout)))
    bias_p = jnp.pad(b.reshape(1, dout), ((0, 0), (0, np_ - dout)))

    cost = pl.CostEstimate(
        flops=2 * rows_p * kp * np_ + 8 * rows_p * kp,
        transcendentals=rows_p,
        bytes_accessed=rows_p * kp * 4 + kp * np_ * 4 + rows_p * np_ * 4,
    )

    out = pl.pallas_call(
        functools.partial(_prenorm_matmul_kernel, eps=eps, true_dim=din),
        out_shape=jax.ShapeDtypeStruct((rows_p, np_), x.dtype),
        grid_spec=pltpu.PrefetchScalarGridSpec(
            num_scalar_prefetch=0,
            grid=(n_cores, inner),
            in_specs=[
                pl.BlockSpec((tm, kp),
                             lambda c, j, inner=inner: (c * inner + j, 0)),
                pl.BlockSpec((1, kp), lambda c, j: (0, 0)),
                pl.BlockSpec((1, kp), lambda c, j: (0, 0)),
                pl.BlockSpec((kp, np_), lambda c, j: (0, 0)),
                pl.BlockSpec((1, np_), lambda c, j: (0, 0)),
            ],
            out_specs=pl.BlockSpec(
                (tm, np_), lambda c, j, inner=inner: (c * inner + j, 0)),
        ),
        compiler_params=pltpu.CompilerParams(
            dimension_semantics=("parallel", "arbitrary"),
            vmem_limit_bytes=60 * 1024 * 1024,
        ),
        cost_estimate=cost,
    )(x_p, g_p, b_p, w_p, bias_p)
    return out[:rows, :dout].reshape(orig_shape[:-1] + (dout,))


# speedup vs baseline: 1.1936x; 1.0048x over previous
import functools

import jax
import jax.numpy as jnp
from jax import lax
from jax.experimental import pallas as pl
from jax.experimental.pallas import tpu as pltpu


def _round_up(x, m):
    return ((x + m - 1) // m) * m


def _prenorm_matmul_kernel(x_ref, g_ref, b_ref, w_ref, bias_ref, o_ref,
                           *, eps, true_dim):
    x = x_ref[...].astype(jnp.float32)
    kp = x.shape[-1]
    inv_d = 1.0 / float(true_dim)
    mean = jnp.sum(x, axis=-1, keepdims=True) * inv_d
    xc = x - mean
    if kp != true_dim:
        mask = lax.broadcasted_iota(jnp.int32, (1, kp), 1) < true_dim
        xc = jnp.where(mask, xc, 0.0)
    var = jnp.sum(xc * xc, axis=-1, keepdims=True) * inv_d
    inv = lax.rsqrt(var + eps)
    y = xc * inv * g_ref[...].astype(jnp.float32) + b_ref[...].astype(jnp.float32)
    y = y.astype(jnp.bfloat16)
    acc = jnp.dot(y, w_ref[...].astype(jnp.bfloat16),
                  preferred_element_type=jnp.float32)
    o_ref[...] = (acc + bias_ref[...].astype(jnp.float32)).astype(o_ref.dtype)


def kernel(x, gamma, beta, w, b):
    eps = 1e-5
    orig_shape = x.shape
    din = orig_shape[-1]
    dout = w.shape[1]
    x2 = x.reshape(-1, din)
    rows = x2.shape[0]

    kp = max(_round_up(din, 128), 128)
    np_ = max(_round_up(dout, 128), 128)

    tm = min(512, _round_up(rows, 8))
    rows_p = _round_up(rows, tm)
    m_tiles = rows_p // tm

    x_p = jnp.pad(x2, ((0, rows_p - rows), (0, kp - din)))
    g_p = jnp.pad(gamma.reshape(1, din), ((0, 0), (0, kp - din)))
    b_p = jnp.pad(beta.reshape(1, din), ((0, 0), (0, kp - din)))
    w_p = jnp.pad(w, ((0, kp - din), (0, np_ - dout)))
    bias_p = jnp.pad(b.reshape(1, dout), ((0, 0), (0, np_ - dout)))

    cost = pl.CostEstimate(
        flops=2 * rows_p * kp * np_ + 8 * rows_p * kp,
        transcendentals=rows_p,
        bytes_accessed=rows_p * kp * 4 + kp * np_ * 4 + rows_p * np_ * 4,
    )

    out = pl.pallas_call(
        functools.partial(_prenorm_matmul_kernel, eps=eps, true_dim=din),
        out_shape=jax.ShapeDtypeStruct((rows_p, np_), x.dtype),
        grid_spec=pltpu.PrefetchScalarGridSpec(
            num_scalar_prefetch=0,
            grid=(m_tiles,),
            in_specs=[
                pl.BlockSpec((tm, kp), lambda i: (i, 0)),
                pl.BlockSpec((1, kp), lambda i: (0, 0)),
                pl.BlockSpec((1, kp), lambda i: (0, 0)),
                pl.BlockSpec((kp, np_), lambda i: (0, 0)),
                pl.BlockSpec((1, np_), lambda i: (0, 0)),
            ],
            out_specs=pl.BlockSpec((tm, np_), lambda i: (i, 0)),
        ),
        compiler_params=pltpu.CompilerParams(
            dimension_semantics=("parallel",),
            vmem_limit_bytes=60 * 1024 * 1024,
        ),
        cost_estimate=cost,
    )(x_p, g_p, b_p, w_p, bias_p)
    return out[:rows, :dout].reshape(orig_shape[:-1] + (dout,))
